# baseline (speedup 1.0000x reference)
import functools

import jax
import jax.numpy as jnp
from jax import lax
from jax.experimental import pallas as pl
from jax.experimental.pallas import tpu as pltpu
from jax.experimental.pallas import tpu_sc as plsc

_BATCH = 16384
_DIM = 64
_NC = 2
_NS = 16
_NW = _NC * _NS
_BPW = _BATCH // _NW
_CHUNK = 128
_NCHUNK = _BPW // _CHUNK
_L = 16
_NGROUP = _BPW // _L
_GPC = _CHUNK // _L


def _sc_body(uvec_hbm, ivec_hbm, uemb_hbm, iemb_hbm, out_hbm,
             uidx, iidx, urows, irows, outv, sem):
    wid = lax.axis_index("s") * _NC + lax.axis_index("c")
    base = wid * _BPW

    for j in range(_NCHUNK):
        pltpu.sync_copy(uvec_hbm.at[pl.ds(base + j * _CHUNK, _CHUNK)], uidx.at[j])
        pltpu.sync_copy(ivec_hbm.at[pl.ds(base + j * _CHUNK, _CHUNK)], iidx.at[j])

    copies = []
    for j in range(_NCHUNK):
        copies.append(pltpu.async_copy(uemb_hbm.at[uidx.at[j]], urows.at[j], sem))
        copies.append(pltpu.async_copy(iemb_hbm.at[iidx.at[j]], irows.at[j], sem))
    for c in copies:
        c.wait()

    iota = jax.lax.iota(jnp.int32, _L)
    for g in range(_NGROUP):
        cu = jnp.full((_L,), g // _GPC, jnp.int32)
        rows = (g % _GPC) * _L + iota

        def body(d, acc):
            dv = jnp.full((_L,), d, jnp.int32)
            uu = plsc.load_gather(urows, [cu, rows, dv])
            vv = plsc.load_gather(irows, [cu, rows, dv])
            return acc + uu * vv

        acc = lax.fori_loop(0, _DIM, body, jnp.zeros((_L,), jnp.float32))
        outv[pl.ds(g * _L, _L)] = 1.0 / (1.0 + jnp.exp(-acc))

    pltpu.sync_copy(outv, out_hbm.at[pl.ds(base, _BPW)])


@functools.partial(jax.jit, donate_argnums=())
def _run(users_vec, items_vec, users_emb, items_emb):
    k = functools.partial(
        pl.kernel,
        mesh=plsc.VectorSubcoreMesh(core_axis_name="c", subcore_axis_name="s"),
        out_type=jax.ShapeDtypeStruct((_BATCH,), jnp.float32),
        compiler_params=pltpu.CompilerParams(
            needs_layout_passes=False,
            use_tc_tiling_on_sc=False,
        ),
        scratch_types=[
            pltpu.VMEM((_NCHUNK, _CHUNK), jnp.int32),
            pltpu.VMEM((_NCHUNK, _CHUNK), jnp.int32),
            pltpu.VMEM((_NCHUNK, _CHUNK, _DIM), jnp.float32),
            pltpu.VMEM((_NCHUNK, _CHUNK, _DIM), jnp.float32),
            pltpu.VMEM((_BPW,), jnp.float32),
            pltpu.SemaphoreType.DMA,
        ],
    )(_sc_body)
    return k(users_vec, items_vec, users_emb, items_emb)


def kernel(users_vec, items_vec, users_emb, items_emb):
    return _run(users_vec, items_vec, users_emb, items_emb)


# speedup vs baseline: 1.5625x; 1.5625x over previous
import functools

import jax
import jax.numpy as jnp
from jax import lax
from jax.experimental import pallas as pl
from jax.experimental.pallas import tpu as pltpu
from jax.experimental.pallas import tpu_sc as plsc

_BATCH = 16384
_DIM = 64
_NC = 2
_NS = 16
_NW = _NC * _NS
_BPW = _BATCH // _NW
_CH = 64
_NCH = _BPW // _CH
_L = 16
_GPC = _CH // _L


def _sc_body(uvec_hbm, ivec_hbm, uemb_hbm, iemb_hbm, out_hbm,
             uixv, iixv, ub0, ub1, ib0, ib1, outv,
             sem_u0, sem_u1, sem_i0, sem_i1):
    wid = lax.axis_index("s") * _NC + lax.axis_index("c")
    base = wid * _BPW

    pltpu.sync_copy(uvec_hbm.at[pl.ds(base, _BPW)], uixv)
    pltpu.sync_copy(ivec_hbm.at[pl.ds(base, _BPW)], iixv)

    ubufs = (ub0, ub1)
    ibufs = (ib0, ib1)
    usems = (sem_u0, sem_u1)
    isems = (sem_i0, sem_i1)

    def fire(c):
        par = c % 2
        ub, ib = ubufs[par], ibufs[par]

        def body(q, carry):
            v_u = uixv[pl.ds(c * _CH + q * _L, _L)]
            v_i = iixv[pl.ds(c * _CH + q * _L, _L)]
            for l in range(_L):
                t = q * _L + l
                pltpu.async_copy(uemb_hbm.at[v_u[l]], ub.at[t], usems[par])
                pltpu.async_copy(iemb_hbm.at[v_i[l]], ib.at[t], isems[par])
            return carry

        lax.fori_loop(0, _GPC, body, 0)

    def drain(c):
        par = c % 2
        dummy = uemb_hbm.at[pl.ds(0, _CH)]
        pltpu.make_async_copy(dummy, ubufs[par], usems[par]).wait()
        pltpu.make_async_copy(dummy, ibufs[par], isems[par]).wait()

    iota = lax.iota(jnp.int32, _L)

    def compute(c):
        par = c % 2
        ub, ib = ubufs[par], ibufs[par]
        for g in range(_GPC):
            rows = g * _L + iota

            def body(d, acc):
                dv = jnp.full((_L,), d, jnp.int32)
                uu = plsc.load_gather(ub, [rows, dv])
                vv = plsc.load_gather(ib, [rows, dv])
                return acc + uu * vv

            acc = lax.fori_loop(0, _DIM, body, jnp.zeros((_L,), jnp.float32))
            outv[pl.ds(c * _CH + g * _L, _L)] = 1.0 / (1.0 + jnp.exp(-acc))

    fire(0)
    for c in range(_NCH):
        if c + 1 < _NCH:
            fire(c + 1)
        drain(c)
        compute(c)

    pltpu.sync_copy(outv, out_hbm.at[pl.ds(base, _BPW)])


@jax.jit
def _run(users_vec, items_vec, users_emb, items_emb):
    k = functools.partial(
        pl.kernel,
        mesh=plsc.VectorSubcoreMesh(core_axis_name="c", subcore_axis_name="s"),
        out_type=jax.ShapeDtypeStruct((_BATCH,), jnp.float32),
        compiler_params=pltpu.CompilerParams(
            needs_layout_passes=False,
        ),
        scratch_types=[
            pltpu.VMEM((_BPW,), jnp.int32),
            pltpu.VMEM((_BPW,), jnp.int32),
            pltpu.VMEM((_CH, _DIM), jnp.float32),
            pltpu.VMEM((_CH, _DIM), jnp.float32),
            pltpu.VMEM((_CH, _DIM), jnp.float32),
            pltpu.VMEM((_CH, _DIM), jnp.float32),
            pltpu.VMEM((_BPW,), jnp.float32),
            pltpu.SemaphoreType.DMA,
            pltpu.SemaphoreType.DMA,
            pltpu.SemaphoreType.DMA,
            pltpu.SemaphoreType.DMA,
        ],
    )(_sc_body)
    return k(users_vec, items_vec, users_emb, items_emb)


def kernel(users_vec, items_vec, users_emb, items_emb):
    return _run(users_vec, items_vec, users_emb, items_emb)


# speedup vs baseline: 1.5636x; 1.0007x over previous
import functools

import jax
import jax.numpy as jnp
from jax import lax
from jax.experimental import pallas as pl
from jax.experimental.pallas import tpu as pltpu
from jax.experimental.pallas import tpu_sc as plsc

_BATCH = 16384
_DIM = 64
_NC = 2
_NS = 16
_NW = _NC * _NS
_BPW = _BATCH // _NW
_CH = 64
_NCH = _BPW // _CH
_L = 16
_GPC = _CH // _L


def _sc_body(uvec_hbm, ivec_hbm, uemb_hbm, iemb_hbm, out_hbm,
             uixv, iixv, ub0, ub1, ib0, ib1, outv, *sems):
    wid = lax.axis_index("s") * _NC + lax.axis_index("c")
    base = wid * _BPW

    pltpu.sync_copy(uvec_hbm.at[pl.ds(base, _BPW)], uixv)
    pltpu.sync_copy(ivec_hbm.at[pl.ds(base, _BPW)], iixv)

    ubufs = (ub0, ub1)
    ibufs = (ib0, ib1)
    usems = (sems[0:4], sems[4:8])
    isems = (sems[8:12], sems[12:16])

    def fire(c):
        par = c % 2
        ub, ib = ubufs[par], ibufs[par]

        def body(q, carry):
            v_u = uixv[pl.ds(c * _CH + q * _L, _L)]
            v_i = iixv[pl.ds(c * _CH + q * _L, _L)]
            for l in range(_L):
                t = q * _L + l
                pltpu.async_copy(uemb_hbm.at[v_u[l]], ub.at[t],
                                 usems[par][l % 4])
                pltpu.async_copy(iemb_hbm.at[v_i[l]], ib.at[t],
                                 isems[par][l % 4])
            return carry

        lax.fori_loop(0, _GPC, body, 0)

    def drain(c):
        par = c % 2
        dummy = uemb_hbm.at[pl.ds(0, _CH // 4)]
        for k in range(4):
            pltpu.make_async_copy(
                dummy, ubufs[par].at[pl.ds(0, _CH // 4)], usems[par][k]).wait()
            pltpu.make_async_copy(
                dummy, ibufs[par].at[pl.ds(0, _CH // 4)], isems[par][k]).wait()

    iota = lax.iota(jnp.int32, _L)

    def compute(c):
        par = c % 2
        ub, ib = ubufs[par], ibufs[par]
        for g in range(_GPC):
            rows = g * _L + iota

            def body(d, acc):
                dv = jnp.full((_L,), d, jnp.int32)
                uu = plsc.load_gather(ub, [rows, dv])
                vv = plsc.load_gather(ib, [rows, dv])
                return acc + uu * vv

            acc = lax.fori_loop(0, _DIM, body, jnp.zeros((_L,), jnp.float32))
            outv[pl.ds(c * _CH + g * _L, _L)] = 1.0 / (1.0 + jnp.exp(-acc))

    fire(0)
    for c in range(_NCH):
        if c + 1 < _NCH:
            fire(c + 1)
        drain(c)
        compute(c)

    pltpu.sync_copy(outv, out_hbm.at[pl.ds(base, _BPW)])


@jax.jit
def _run(users_vec, items_vec, users_emb, items_emb):
    k = functools.partial(
        pl.kernel,
        mesh=plsc.VectorSubcoreMesh(core_axis_name="c", subcore_axis_name="s"),
        out_type=jax.ShapeDtypeStruct((_BATCH,), jnp.float32),
        compiler_params=pltpu.CompilerParams(
            needs_layout_passes=False,
        ),
        scratch_types=[
            pltpu.VMEM((_BPW,), jnp.int32),
            pltpu.VMEM((_BPW,), jnp.int32),
            pltpu.VMEM((_CH, _DIM), jnp.float32),
            pltpu.VMEM((_CH, _DIM), jnp.float32),
            pltpu.VMEM((_CH, _DIM), jnp.float32),
            pltpu.VMEM((_CH, _DIM), jnp.float32),
            pltpu.VMEM((_BPW,), jnp.float32),
        ] + [pltpu.SemaphoreType.DMA] * 16,
    )(_sc_body)
    return k(users_vec, items_vec, users_emb, items_emb)


def kernel(users_vec, items_vec, users_emb, items_emb):
    return _run(users_vec, items_vec, users_emb, items_emb)
